# Initial kernel scaffold; baseline (speedup 1.0000x reference)
import jax
import jax.numpy as jnp
from jax.experimental import pallas as pl


def kernel(x, table):
    raise NotImplementedError("write your pallas kernel here")


# speedup vs baseline: 1.4942x; 1.4942x over previous
import functools

import jax
import jax.numpy as jnp
from jax import lax
from jax.experimental import pallas as pl
from jax.experimental.pallas import tpu as pltpu
from jax.experimental.pallas import tpu_sc as plsc

_NC = 2
_NS = 16
_NW = _NC * _NS

_CHUNK = 3200


@functools.partial(jax.jit, static_argnums=(2, 3))
def _sc_gather(idx, table, B, D):
    n_chunks = B // (_NW * _CHUNK)
    mesh = plsc.VectorSubcoreMesh(core_axis_name="c", subcore_axis_name="s")

    @functools.partial(
        pl.kernel,
        mesh=mesh,
        out_type=jax.ShapeDtypeStruct((B, D), jnp.float32),
        scratch_types=[
            pltpu.VMEM((_CHUNK,), jnp.int32),
            pltpu.VMEM((_CHUNK, D), jnp.float32),
            pltpu.SemaphoreType.DMA,
        ],
        compiler_params=pltpu.CompilerParams(use_tc_tiling_on_sc=False),
    )
    def k(idx_hbm, table_hbm, out_hbm, idx_v, rows_v, sem):
        wid = lax.axis_index("s") * _NC + lax.axis_index("c")
        base = wid * (n_chunks * _CHUNK)

        def body(i, carry):
            off = base + i * _CHUNK
            pltpu.sync_copy(idx_hbm.at[pl.ds(off, _CHUNK)], idx_v)
            pltpu.async_copy(table_hbm.at[idx_v], rows_v, sem).wait()
            pltpu.sync_copy(rows_v, out_hbm.at[pl.ds(off, _CHUNK)])
            return carry

        lax.fori_loop(0, n_chunks, body, 0)

    return k(idx, table)


def kernel(x, table):
    D = table.shape[1]
    idx = x.reshape(-1).astype(jnp.int32)
    B = idx.shape[0]
    out = _sc_gather(idx, table, B, D)
    return out.reshape(x.shape + (D,))


# speedup vs baseline: 1.5000x; 1.0038x over previous
import functools

import jax
import jax.numpy as jnp
from jax import lax
from jax.experimental import pallas as pl
from jax.experimental.pallas import tpu as pltpu
from jax.experimental.pallas import tpu_sc as plsc

_NC = 2
_NS = 16
_NW = _NC * _NS

_CHUNK = 1280


@functools.partial(jax.jit, static_argnums=(2, 3))
def _sc_gather(idx, table, B, D):
    bpw = B // _NW
    n_chunks = bpw // _CHUNK
    mesh = plsc.VectorSubcoreMesh(core_axis_name="c", subcore_axis_name="s")

    @functools.partial(
        pl.kernel,
        mesh=mesh,
        out_type=jax.ShapeDtypeStruct((B, D), jnp.float32),
        scratch_types=[
            pltpu.VMEM((bpw,), jnp.int32),
            pltpu.VMEM((_CHUNK, D), jnp.float32),
            pltpu.VMEM((_CHUNK, D), jnp.float32),
            pltpu.SemaphoreType.DMA((2,)),
            pltpu.SemaphoreType.DMA((2,)),
        ],
        compiler_params=pltpu.CompilerParams(use_tc_tiling_on_sc=False),
    )
    def k(idx_hbm, table_hbm, out_hbm, idx_v, rows0, rows1, gsem, wsem):
        wid = lax.axis_index("s") * _NC + lax.axis_index("c")
        base = wid * bpw
        bufs = (rows0, rows1)

        pltpu.sync_copy(idx_hbm.at[pl.ds(base, bpw)], idx_v)

        def gather_start(i, b):
            pltpu.make_async_copy(
                table_hbm.at[idx_v.at[pl.ds(i * _CHUNK, _CHUNK)]],
                bufs[b], gsem.at[b]).start()

        def gather_wait(b):
            pltpu.make_async_copy(
                table_hbm.at[idx_v.at[pl.ds(0, _CHUNK)]],
                bufs[b], gsem.at[b]).wait()

        def wb_start(i, b):
            pltpu.make_async_copy(
                bufs[b], out_hbm.at[pl.ds(base + i * _CHUNK, _CHUNK)],
                wsem.at[b]).start()

        def wb_wait(b):
            pltpu.make_async_copy(
                bufs[b], out_hbm.at[pl.ds(base, _CHUNK)],
                wsem.at[b]).wait()

        gather_start(0, 0)
        for i in range(n_chunks):
            b = i & 1
            nb = 1 - b
            if i + 1 < n_chunks:
                if i >= 1:
                    wb_wait(nb)
                gather_start(i + 1, nb)
            gather_wait(b)
            wb_start(i, b)
        wb_wait((n_chunks - 1) & 1)
        wb_wait((n_chunks - 2) & 1)

    return k(idx, table)


def kernel(x, table):
    D = table.shape[1]
    idx = x.reshape(-1).astype(jnp.int32)
    B = idx.shape[0]
    out = _sc_gather(idx, table, B, D)
    return out.reshape(x.shape + (D,))
